# baseline (speedup 1.0000x reference)
import functools

import jax
import jax.numpy as jnp
from jax import lax
from jax.experimental import pallas as pl
from jax.experimental.pallas import tpu as pltpu
from jax.experimental.pallas import tpu_sc as plsc

B = 16
T = 2048
D = 128
H = 8
NB = 32
BS = 64
S = H * T
NCHUNK = S // BS
G = B * NCHUNK
ROWS = B * S
NW = 32
RPW = ROWS // NW
CH = 128
SELF_ATTN_VALUE = -50000.0


def _hash_body(qk_ref, r_ref, out_ref):
    x = qk_ref[0]
    r = r_ref[...]
    rot = jnp.dot(x, r, preferred_element_type=jnp.float32)
    for h in range(H):
        rh = rot[:, h * 16:(h + 1) * 16]
        c = jnp.concatenate([rh, -rh], axis=1)
        m = jnp.max(c, axis=1, keepdims=True)
        iota = lax.broadcasted_iota(jnp.int32, (T, NB), 1)
        idx = jnp.min(jnp.where(c >= m, iota, NB), axis=1, keepdims=True)
        out_ref[0, :, h * 16:(h + 1) * 16] = jnp.broadcast_to(
            idx + h * NB, (T, 16))


def _hash_buckets(qk, rmat):
    out = pl.pallas_call(
        _hash_body,
        grid=(B,),
        in_specs=[
            pl.BlockSpec((1, T, D), lambda b: (b, 0, 0)),
            pl.BlockSpec((D, 128), lambda b: (0, 0)),
        ],
        out_specs=pl.BlockSpec((1, T, 128), lambda b: (b, 0, 0)),
        out_shape=jax.ShapeDtypeStruct((B, T, 128), jnp.int32),
    )(qk, rmat)
    return out[:, :, ::16]


def _make_sc_gather():
    mesh = plsc.VectorSubcoreMesh(core_axis_name="c", subcore_axis_name="s")

    @functools.partial(
        pl.kernel,
        mesh=mesh,
        out_type=[
            jax.ShapeDtypeStruct((ROWS, D), jnp.float32),
            jax.ShapeDtypeStruct((ROWS, D), jnp.float32),
        ],
        scratch_types=[
            pltpu.VMEM((RPW,), jnp.int32),
            pltpu.VMEM((CH, D), jnp.float32),
            pltpu.VMEM((CH, D), jnp.float32),
            pltpu.SemaphoreType.DMA,
            pltpu.SemaphoreType.DMA,
        ],
    )
    def gather(idx_hbm, qk_hbm, v_hbm, sqk_hbm, sv_hbm,
               idx_v, bufq, bufv, sem1, sem2):
        wid = lax.axis_index("s") * 2 + lax.axis_index("c")
        base = wid * RPW
        pltpu.sync_copy(idx_hbm.at[pl.ds(base, RPW)], idx_v)

        def body(i, carry):
            off = i * CH
            cq = pltpu.async_copy(
                qk_hbm.at[idx_v.at[pl.ds(off, CH)]], bufq, sem1)
            cv = pltpu.async_copy(
                v_hbm.at[idx_v.at[pl.ds(off, CH)]], bufv, sem2)
            cq.wait()
            cv.wait()
            pltpu.sync_copy(bufq, sqk_hbm.at[pl.ds(base + off, CH)])
            pltpu.sync_copy(bufv, sv_hbm.at[pl.ds(base + off, CH)])
            return carry

        lax.fori_loop(0, RPW // CH, body, 0)

    return gather


def _make_sc_unsort():
    mesh = plsc.VectorSubcoreMesh(core_axis_name="c", subcore_axis_name="s")

    @functools.partial(
        pl.kernel,
        mesh=mesh,
        compiler_params=pltpu.CompilerParams(use_tc_tiling_on_sc=False),
        out_type=[
            jax.ShapeDtypeStruct((ROWS, D), jnp.float32),
            jax.ShapeDtypeStruct((ROWS, 16), jnp.float32),
        ],
        scratch_types=[
            pltpu.VMEM((RPW // CH, CH), jnp.int32),
            pltpu.VMEM((RPW // CH, CH), jnp.int32),
            pltpu.VMEM((CH, D), jnp.float32),
            pltpu.VMEM((CH, 16), jnp.float32),
            pltpu.SemaphoreType.DMA,
            pltpu.SemaphoreType.DMA,
        ],
    )
    def unsort(idxo_hbm, idxl_hbm, so_hbm, slog_hbm, otok_hbm, ltok_hbm,
               idxo_v, idxl_v, buf, bufl, sem, seml):
        wid = lax.axis_index("s") * 2 + lax.axis_index("c")
        base = wid * RPW
        pltpu.sync_copy(idxo_hbm.at[wid], idxo_v)
        pltpu.sync_copy(idxl_hbm.at[wid], idxl_v)

        def rows(i, carry):
            off = i * CH
            pltpu.sync_copy(so_hbm.at[pl.ds(base + off, CH)], buf)
            pltpu.sync_copy(slog_hbm.at[pl.ds(base + off, CH)], bufl)
            co = pltpu.async_copy(buf, otok_hbm.at[idxo_v.at[i]], sem)
            cl = pltpu.async_copy(bufl, ltok_hbm.at[idxl_v.at[i]], seml)
            co.wait()
            cl.wait()
            return carry

        lax.fori_loop(0, RPW // CH, rows, 0)

    return unsort


NCH = 8
MB = NCH * BS
GSTEPS = G // NCH


def _attn_body(rows_ref, pedge_ref, idl_ref, idlp_ref, idc_ref,
               vrows_ref, vpedge_ref, so_ref, slog_ref):
    q = rows_ref[0]
    k_all = jnp.concatenate([pedge_ref[0], q], axis=0)
    nrm = jnp.sqrt(jnp.sum(k_all * k_all, axis=1, keepdims=True))
    k_all = k_all / jnp.maximum(nrm, 1e-12)
    dots = lax.dot_general(
        q.astype(jnp.bfloat16), k_all.astype(jnp.bfloat16),
        (((1,), (1,)), ((), ())),
        preferred_element_type=jnp.float32) * (D ** -0.5)

    ci = lax.broadcasted_iota(jnp.int32, (MB, MB + BS), 0) // BS
    cj = lax.broadcasted_iota(jnp.int32, (MB, MB + BS), 1) // BS
    band = (cj == ci) | (cj == ci + 1)

    qid = idc_ref[0]
    kid = jnp.concatenate([idlp_ref[0], idl_ref[0]], axis=1)
    dots = jnp.where(qid == kid, SELF_ATTN_VALUE, dots)
    dots = jnp.where(band, dots, -1e30)

    m = jnp.max(dots, axis=1, keepdims=True)
    e = jnp.exp(dots - m)
    s = jnp.sum(e, axis=1, keepdims=True)
    p = e / s
    v_all = jnp.concatenate([vpedge_ref[0], vrows_ref[0]], axis=0)
    so_ref[0] = jnp.dot(p.astype(jnp.bfloat16), v_all.astype(jnp.bfloat16),
                        preferred_element_type=jnp.float32)
    slog_ref[0] = jnp.broadcast_to(jnp.log(s) + m, (MB, 16))


def _pedge_map(g):
    c0 = g * NCH
    return jnp.where(c0 % NCHUNK == 0, c0 + NCHUNK - 1, c0 - 1)


def _attention(sqk, sv, st):
    rows = sqk.reshape(GSTEPS, MB, D)
    vrows = sv.reshape(GSTEPS, MB, D)
    id_lane = st.reshape(GSTEPS, 1, MB)
    id_lane_c = st.reshape(G, 1, BS)
    id_col = st.reshape(GSTEPS, MB, 1)
    return pl.pallas_call(
        _attn_body,
        grid=(GSTEPS,),
        in_specs=[
            pl.BlockSpec((1, MB, D), lambda g: (g, 0, 0)),
            pl.BlockSpec((1, BS, D), lambda g: (_pedge_map(g), 0, 0)),
            pl.BlockSpec((1, 1, MB), lambda g: (g, 0, 0)),
            pl.BlockSpec((1, 1, BS), lambda g: (_pedge_map(g), 0, 0)),
            pl.BlockSpec((1, MB, 1), lambda g: (g, 0, 0)),
            pl.BlockSpec((1, MB, D), lambda g: (g, 0, 0)),
            pl.BlockSpec((1, BS, D), lambda g: (_pedge_map(g), 0, 0)),
        ],
        out_specs=[
            pl.BlockSpec((1, MB, D), lambda g: (g, 0, 0)),
            pl.BlockSpec((1, MB, 16), lambda g: (g, 0, 0)),
        ],
        out_shape=[
            jax.ShapeDtypeStruct((GSTEPS, MB, D), jnp.float32),
            jax.ShapeDtypeStruct((GSTEPS, MB, 16), jnp.float32),
        ],
    )(rows, sqk, id_lane, id_lane_c, id_col, vrows, sv)


_TB = 256


def _combine_body(o_ref, l_ref, out_ref):
    l = l_ref[0][:, :, 0]
    m = jnp.max(l, axis=1, keepdims=True)
    e = jnp.exp(l - m)
    w = e / jnp.sum(e, axis=1, keepdims=True)
    acc = jnp.zeros((_TB, D), jnp.float32)
    for h in range(H):
        acc = acc + o_ref[0, h] * w[:, h:h + 1]
    out_ref[0] = acc


def _combine(o_tok, l_tok):
    return pl.pallas_call(
        _combine_body,
        grid=(B, T // _TB),
        in_specs=[
            pl.BlockSpec((1, H, _TB, D), lambda b, t: (b, 0, t, 0)),
            pl.BlockSpec((1, _TB, H, 16), lambda b, t: (b, t, 0, 0)),
        ],
        out_specs=pl.BlockSpec((1, _TB, D), lambda b, t: (b, t, 0)),
        out_shape=jax.ShapeDtypeStruct((B, T, D), jnp.float32),
    )(o_tok, l_tok)


@jax.jit
def _run(qk, v):
    rot = jax.random.normal(
        jax.random.key(42), (1, D, H, NB // 2), dtype=jnp.float32)
    rmat = rot[0].reshape(D, H * (NB // 2))

    buckets = _hash_buckets(qk, rmat)
    keys = (buckets.transpose(0, 2, 1) * T
            + jnp.arange(T, dtype=jnp.int32)[None, None, :]).reshape(B, S)
    sort_idx = jnp.argsort(keys, axis=-1).astype(jnp.int32)
    st = (sort_idx % T).astype(jnp.int32)

    gather_idx = (st + jnp.arange(B, dtype=jnp.int32)[:, None] * T
                  ).reshape(ROWS)
    sc_gather = _make_sc_gather()
    sqk, sv = sc_gather(gather_idx, qk.reshape(B * T, D), v.reshape(B * T, D))

    so, slog = _attention(
        sqk.reshape(G, BS, D), sv.reshape(G, BS, D), st.reshape(G, BS))

    harr = (jnp.arange(S, dtype=jnp.int32) // T)[None, :]
    boff = jnp.arange(B, dtype=jnp.int32)[:, None] * S
    dst_o = (harr * T + st + boff).reshape(NW, RPW // CH, CH)
    dst_l = (st * H + harr + boff).reshape(NW, RPW // CH, CH)
    sc_unsort = _make_sc_unsort()
    o_tok, l_tok = sc_unsort(
        dst_o, dst_l, so.reshape(ROWS, D), slog.reshape(ROWS, 16))

    return _combine(o_tok.reshape(B, H, T, D), l_tok.reshape(B, T, H, 16))


def kernel(qk, v):
    return _run(qk, v)


# speedup vs baseline: 1.2284x; 1.2284x over previous
import functools

import jax
import jax.numpy as jnp
from jax import lax
from jax.experimental import pallas as pl
from jax.experimental.pallas import tpu as pltpu
from jax.experimental.pallas import tpu_sc as plsc

B = 16
T = 2048
D = 128
H = 8
NB = 32
BS = 64
S = H * T
NCHUNK = S // BS
G = B * NCHUNK
ROWS = B * S
NW = 32
RPW = ROWS // NW
CH = 128
SELF_ATTN_VALUE = -50000.0


def _hash_body(qkt_ref, r_ref, out_ref):
    x = qkt_ref[0]
    r = r_ref[...]
    rot = jnp.dot(r, x, preferred_element_type=jnp.float32)
    rows = []
    for h in range(H):
        rh = rot[h * 16:(h + 1) * 16]
        a = jnp.maximum(rh, -rh)
        a8 = jnp.maximum(a[0:8], a[8:16])
        m = jnp.max(a8, axis=0, keepdims=True)
        iota = lax.broadcasted_iota(jnp.int32, (16, T), 0)
        candp = jnp.where(rh >= m, iota, NB)
        candn = jnp.where(-rh >= m, iota + 16, NB)
        cand = jnp.minimum(candp, candn)
        c8 = jnp.minimum(cand[0:8], cand[8:16])
        idx = jnp.min(c8, axis=0, keepdims=True)
        rows.append(idx + h * NB)
    bucket = jnp.concatenate(rows, axis=0)
    out_ref[0] = bucket * T + lax.broadcasted_iota(jnp.int32, (H, T), 1)


def _hash_keys(qk_t, rmat_t):
    return pl.pallas_call(
        _hash_body,
        grid=(B,),
        in_specs=[
            pl.BlockSpec((1, D, T), lambda b: (b, 0, 0)),
            pl.BlockSpec((128, D), lambda b: (0, 0)),
        ],
        out_specs=pl.BlockSpec((1, H, T), lambda b: (b, 0, 0)),
        out_shape=jax.ShapeDtypeStruct((B, H, T), jnp.int32),
    )(qk_t, rmat_t)


def _make_sc_gather():
    mesh = plsc.VectorSubcoreMesh(core_axis_name="c", subcore_axis_name="s")

    @functools.partial(
        pl.kernel,
        mesh=mesh,
        out_type=[
            jax.ShapeDtypeStruct((ROWS, D), jnp.float32),
            jax.ShapeDtypeStruct((ROWS, D), jnp.float32),
        ],
        scratch_types=[
            pltpu.VMEM((RPW,), jnp.int32),
            pltpu.VMEM((CH, D), jnp.float32),
            pltpu.VMEM((CH, D), jnp.float32),
            pltpu.SemaphoreType.DMA,
            pltpu.SemaphoreType.DMA,
        ],
    )
    def gather(idx_hbm, qk_hbm, v_hbm, sqk_hbm, sv_hbm,
               idx_v, bufq, bufv, sem1, sem2):
        wid = lax.axis_index("s") * 2 + lax.axis_index("c")
        base = wid * RPW
        pltpu.sync_copy(idx_hbm.at[pl.ds(base, RPW)], idx_v)

        def body(i, carry):
            off = i * CH
            cq = pltpu.async_copy(
                qk_hbm.at[idx_v.at[pl.ds(off, CH)]], bufq, sem1)
            cv = pltpu.async_copy(
                v_hbm.at[idx_v.at[pl.ds(off, CH)]], bufv, sem2)
            cq.wait()
            cv.wait()
            pltpu.sync_copy(bufq, sqk_hbm.at[pl.ds(base + off, CH)])
            pltpu.sync_copy(bufv, sv_hbm.at[pl.ds(base + off, CH)])
            return carry

        lax.fori_loop(0, RPW // CH, body, 0)

    return gather


def _make_sc_unsort():
    mesh = plsc.VectorSubcoreMesh(core_axis_name="c", subcore_axis_name="s")

    @functools.partial(
        pl.kernel,
        mesh=mesh,
        compiler_params=pltpu.CompilerParams(use_tc_tiling_on_sc=False),
        out_type=[
            jax.ShapeDtypeStruct((ROWS, D), jnp.float32),
            jax.ShapeDtypeStruct((ROWS, 16), jnp.float32),
        ],
        scratch_types=[
            pltpu.VMEM((RPW // CH, CH), jnp.int32),
            pltpu.VMEM((RPW // CH, CH), jnp.int32),
            pltpu.VMEM((CH, D), jnp.float32),
            pltpu.VMEM((CH, 16), jnp.float32),
            pltpu.SemaphoreType.DMA,
            pltpu.SemaphoreType.DMA,
        ],
    )
    def unsort(idxo_hbm, idxl_hbm, so_hbm, slog_hbm, otok_hbm, ltok_hbm,
               idxo_v, idxl_v, buf, bufl, sem, seml):
        wid = lax.axis_index("s") * 2 + lax.axis_index("c")
        base = wid * RPW
        pltpu.sync_copy(idxo_hbm.at[wid], idxo_v)
        pltpu.sync_copy(idxl_hbm.at[wid], idxl_v)

        def rows(i, carry):
            off = i * CH
            pltpu.sync_copy(so_hbm.at[pl.ds(base + off, CH)], buf)
            pltpu.sync_copy(slog_hbm.at[pl.ds(base + off, CH)], bufl)
            co = pltpu.async_copy(buf, otok_hbm.at[idxo_v.at[i]], sem)
            cl = pltpu.async_copy(bufl, ltok_hbm.at[idxl_v.at[i]], seml)
            co.wait()
            cl.wait()
            return carry

        lax.fori_loop(0, RPW // CH, rows, 0)

    return unsort


NCH = 8
MB = NCH * BS
GSTEPS = G // NCH


def _attn_body(rows_ref, pedge_ref, idl_ref, idlp_ref, idc_ref,
               vrows_ref, vpedge_ref, so_ref, slog_ref):
    rows = rows_ref[0]
    k_all = jnp.concatenate([pedge_ref[0], rows], axis=0)
    nrm = jnp.sqrt(jnp.sum(k_all * k_all, axis=1, keepdims=True))
    k_all = k_all / jnp.maximum(nrm, 1e-12)

    q8 = rows.reshape(NCH, BS, D)
    k_cat = jnp.concatenate(
        [k_all[BS:].reshape(NCH, BS, D),
         k_all[:MB].reshape(NCH, BS, D)], axis=1)
    dots = lax.dot_general(
        q8.astype(jnp.bfloat16), k_cat.astype(jnp.bfloat16),
        (((2,), (2,)), ((0,), (0,))),
        preferred_element_type=jnp.float32) * (D ** -0.5)

    qid = idc_ref[0].reshape(NCH, BS, 1)
    kid = jnp.concatenate([idl_ref[:, 0], idlp_ref[:, 0]],
                          axis=1).reshape(NCH, 1, 2 * BS)
    dots = jnp.where(qid == kid, SELF_ATTN_VALUE, dots)

    m = jnp.max(dots, axis=2, keepdims=True)
    e = jnp.exp(dots - m)
    s = jnp.sum(e, axis=2, keepdims=True)
    p = e / s

    v_all = jnp.concatenate([vpedge_ref[0], vrows_ref[0]], axis=0)
    v_cat = jnp.concatenate(
        [v_all[BS:].reshape(NCH, BS, D),
         v_all[:MB].reshape(NCH, BS, D)], axis=1)
    bo = lax.dot_general(
        p.astype(jnp.bfloat16), v_cat.astype(jnp.bfloat16),
        (((2,), (1,)), ((0,), (0,))),
        preferred_element_type=jnp.float32)
    so_ref[0] = bo.reshape(MB, D)
    slog_ref[0] = jnp.broadcast_to(
        (jnp.log(s) + m).reshape(MB, 1), (MB, 16))


def _pedge_map(g):
    c0 = g * NCH
    return jnp.where(c0 % NCHUNK == 0, c0 + NCHUNK - 1, c0 - 1)


def _attention(sqk, sv, st, st_prev):
    rows = sqk.reshape(GSTEPS, MB, D)
    vrows = sv.reshape(GSTEPS, MB, D)
    id_lane_c = st.reshape(G, 1, BS)
    id_lane_p = st_prev.reshape(G, 1, BS)
    id_col = st.reshape(GSTEPS, MB, 1)
    return pl.pallas_call(
        _attn_body,
        grid=(GSTEPS,),
        in_specs=[
            pl.BlockSpec((1, MB, D), lambda g: (g, 0, 0)),
            pl.BlockSpec((1, BS, D), lambda g: (_pedge_map(g), 0, 0)),
            pl.BlockSpec((NCH, 1, BS), lambda g: (g, 0, 0)),
            pl.BlockSpec((NCH, 1, BS), lambda g: (g, 0, 0)),
            pl.BlockSpec((1, MB, 1), lambda g: (g, 0, 0)),
            pl.BlockSpec((1, MB, D), lambda g: (g, 0, 0)),
            pl.BlockSpec((1, BS, D), lambda g: (_pedge_map(g), 0, 0)),
        ],
        out_specs=[
            pl.BlockSpec((1, MB, D), lambda g: (g, 0, 0)),
            pl.BlockSpec((1, MB, 16), lambda g: (g, 0, 0)),
        ],
        out_shape=[
            jax.ShapeDtypeStruct((GSTEPS, MB, D), jnp.float32),
            jax.ShapeDtypeStruct((GSTEPS, MB, 16), jnp.float32),
        ],
    )(rows, sqk, id_lane_c, id_lane_p, id_col, vrows, sv)


_TB = 256


def _combine_body(o_ref, l_ref, out_ref):
    l = l_ref[0][:, :, 0]
    m = jnp.max(l, axis=1, keepdims=True)
    e = jnp.exp(l - m)
    w = e / jnp.sum(e, axis=1, keepdims=True)
    acc = jnp.zeros((_TB, D), jnp.float32)
    for h in range(H):
        acc = acc + o_ref[0, h] * w[:, h:h + 1]
    out_ref[0] = acc


def _combine(o_tok, l_tok):
    return pl.pallas_call(
        _combine_body,
        grid=(B, T // _TB),
        in_specs=[
            pl.BlockSpec((1, H, _TB, D), lambda b, t: (b, 0, t, 0)),
            pl.BlockSpec((1, _TB, H, 16), lambda b, t: (b, t, 0, 0)),
        ],
        out_specs=pl.BlockSpec((1, _TB, D), lambda b, t: (b, t, 0)),
        out_shape=jax.ShapeDtypeStruct((B, T, D), jnp.float32),
    )(o_tok, l_tok)


@jax.jit
def _run(qk, v):
    rot = jax.random.normal(
        jax.random.key(42), (1, D, H, NB // 2), dtype=jnp.float32)
    rmat_t = rot[0].reshape(D, H * (NB // 2)).T

    keys = _hash_keys(qk.transpose(0, 2, 1), rmat_t).reshape(B, S)
    sort_idx = jnp.argsort(keys, axis=-1).astype(jnp.int32)
    st = (sort_idx % T).astype(jnp.int32)

    gather_idx = (st + jnp.arange(B, dtype=jnp.int32)[:, None] * T
                  ).reshape(ROWS)
    sc_gather = _make_sc_gather()
    sqk, sv = sc_gather(gather_idx, qk.reshape(B * T, D), v.reshape(B * T, D))

    st_prev = jnp.roll(st, BS, axis=1)
    so, slog = _attention(
        sqk.reshape(G, BS, D), sv.reshape(G, BS, D),
        st.reshape(G, BS), st_prev.reshape(G, BS))

    harr = (jnp.arange(S, dtype=jnp.int32) // T)[None, :]
    boff = jnp.arange(B, dtype=jnp.int32)[:, None] * S
    dst_o = (harr * T + st + boff).reshape(NW, RPW // CH, CH)
    dst_l = (st * H + harr + boff).reshape(NW, RPW // CH, CH)
    sc_unsort = _make_sc_unsort()
    o_tok, l_tok = sc_unsort(
        dst_o, dst_l, so.reshape(ROWS, D), slog.reshape(ROWS, 16))

    return _combine(o_tok.reshape(B, H, T, D), l_tok.reshape(B, T, H, 16))


def kernel(qk, v):
    return _run(qk, v)


# speedup vs baseline: 1.3915x; 1.1328x over previous
import functools

import jax
import jax.numpy as jnp
from jax import lax
from jax.experimental import pallas as pl
from jax.experimental.pallas import tpu as pltpu
from jax.experimental.pallas import tpu_sc as plsc

B = 16
T = 2048
D = 128
H = 8
NB = 32
BS = 64
S = H * T
NCHUNK = S // BS
G = B * NCHUNK
ROWS = B * S
NW = 32
RPW = ROWS // NW
CH = 128
SELF_ATTN_VALUE = -50000.0


def _hash_body(qkt_ref, r_ref, out_ref):
    x = qkt_ref[0]
    r = r_ref[...]
    rot = jnp.dot(r, x, preferred_element_type=jnp.float32)
    rows = []
    for h in range(H):
        rh = rot[h * 16:(h + 1) * 16]
        a = jnp.maximum(rh, -rh)
        a8 = jnp.maximum(a[0:8], a[8:16])
        m = jnp.max(a8, axis=0, keepdims=True)
        iota = lax.broadcasted_iota(jnp.int32, (16, T), 0)
        candp = jnp.where(rh >= m, iota, NB)
        candn = jnp.where(-rh >= m, iota + 16, NB)
        cand = jnp.minimum(candp, candn)
        c8 = jnp.minimum(cand[0:8], cand[8:16])
        idx = jnp.min(c8, axis=0, keepdims=True)
        rows.append(idx + h * NB)
    bucket = jnp.concatenate(rows, axis=0)
    out_ref[0] = bucket * T + lax.broadcasted_iota(jnp.int32, (H, T), 1)


def _hash_keys(qk_t, rmat_t):
    return pl.pallas_call(
        _hash_body,
        grid=(B,),
        in_specs=[
            pl.BlockSpec((1, D, T), lambda b: (b, 0, 0)),
            pl.BlockSpec((128, D), lambda b: (0, 0)),
        ],
        out_specs=pl.BlockSpec((1, H, T), lambda b: (b, 0, 0)),
        out_shape=jax.ShapeDtypeStruct((B, H, T), jnp.int32),
    )(qk_t, rmat_t)


def _make_sc_gather():
    mesh = plsc.VectorSubcoreMesh(core_axis_name="c", subcore_axis_name="s")

    @functools.partial(
        pl.kernel,
        mesh=mesh,
        out_type=[
            jax.ShapeDtypeStruct((ROWS, D), jnp.float32),
            jax.ShapeDtypeStruct((ROWS, D), jnp.float32),
        ],
        scratch_types=[
            pltpu.VMEM((RPW,), jnp.int32),
            pltpu.VMEM((CH, D), jnp.float32),
            pltpu.VMEM((CH, D), jnp.float32),
            pltpu.SemaphoreType.DMA,
            pltpu.SemaphoreType.DMA,
        ],
    )
    def gather(idx_hbm, qk_hbm, v_hbm, sqk_hbm, sv_hbm,
               idx_v, bufq, bufv, sem1, sem2):
        wid = lax.axis_index("s") * 2 + lax.axis_index("c")
        base = wid * RPW
        pltpu.sync_copy(idx_hbm.at[pl.ds(base, RPW)], idx_v)

        def body(i, carry):
            off = i * CH
            cq = pltpu.async_copy(
                qk_hbm.at[idx_v.at[pl.ds(off, CH)]], bufq, sem1)
            cv = pltpu.async_copy(
                v_hbm.at[idx_v.at[pl.ds(off, CH)]], bufv, sem2)
            cq.wait()
            cv.wait()
            pltpu.sync_copy(bufq, sqk_hbm.at[pl.ds(base + off, CH)])
            pltpu.sync_copy(bufv, sv_hbm.at[pl.ds(base + off, CH)])
            return carry

        lax.fori_loop(0, RPW // CH, body, 0)

    return gather


def _make_sc_unsort():
    mesh = plsc.VectorSubcoreMesh(core_axis_name="c", subcore_axis_name="s")

    @functools.partial(
        pl.kernel,
        mesh=mesh,
        compiler_params=pltpu.CompilerParams(use_tc_tiling_on_sc=False),
        out_type=[
            jax.ShapeDtypeStruct((ROWS, D), jnp.float32),
            jax.ShapeDtypeStruct((ROWS, 16), jnp.float32),
        ],
        scratch_types=[
            pltpu.VMEM((RPW // CH, CH), jnp.int32),
            pltpu.VMEM((RPW // CH, CH), jnp.int32),
            pltpu.VMEM((CH, D), jnp.float32),
            pltpu.VMEM((CH, 16), jnp.float32),
            pltpu.SemaphoreType.DMA,
            pltpu.SemaphoreType.DMA,
        ],
    )
    def unsort(idxo_hbm, idxl_hbm, so_hbm, slog_hbm, otok_hbm, ltok_hbm,
               idxo_v, idxl_v, buf, bufl, sem, seml):
        wid = lax.axis_index("s") * 2 + lax.axis_index("c")
        base = wid * RPW
        pltpu.sync_copy(idxo_hbm.at[wid], idxo_v)
        pltpu.sync_copy(idxl_hbm.at[wid], idxl_v)

        def rows(i, carry):
            off = i * CH
            pltpu.sync_copy(so_hbm.at[pl.ds(base + off, CH)], buf)
            pltpu.sync_copy(slog_hbm.at[pl.ds(base + off, CH)], bufl)
            co = pltpu.async_copy(buf, otok_hbm.at[idxo_v.at[i]], sem)
            cl = pltpu.async_copy(bufl, ltok_hbm.at[idxl_v.at[i]], seml)
            co.wait()
            cl.wait()
            return carry

        lax.fori_loop(0, RPW // CH, rows, 0)

    return unsort


NCH = 8
MB = NCH * BS
GSTEPS = G // NCH


def _attn_body(rows_ref, pedge_ref, idl_ref, idlp_ref, idc_ref,
               vrows_ref, vpedge_ref, so_ref, slog_ref):
    rows = rows_ref[0]
    k_all = jnp.concatenate([pedge_ref[0], rows], axis=0)
    nrm = jnp.sqrt(jnp.sum(k_all * k_all, axis=1, keepdims=True))
    k_all = k_all / jnp.maximum(nrm, 1e-12)

    q8 = rows.reshape(NCH, BS, D)
    k_cat = jnp.concatenate(
        [k_all[BS:].reshape(NCH, BS, D),
         k_all[:MB].reshape(NCH, BS, D)], axis=1)
    dots = lax.dot_general(
        q8.astype(jnp.bfloat16), k_cat.astype(jnp.bfloat16),
        (((2,), (2,)), ((0,), (0,))),
        preferred_element_type=jnp.float32) * (D ** -0.5)

    qid = idc_ref[0].reshape(NCH, BS, 1)
    kid = jnp.concatenate([idl_ref[:, 0], idlp_ref[:, 0]],
                          axis=1).reshape(NCH, 1, 2 * BS)
    dots = jnp.where(qid == kid, SELF_ATTN_VALUE, dots)

    m = jnp.max(dots, axis=2, keepdims=True)
    e = jnp.exp(dots - m)
    s = jnp.sum(e, axis=2, keepdims=True)
    p = e / s

    v_all = jnp.concatenate([vpedge_ref[0], vrows_ref[0]], axis=0)
    v_cat = jnp.concatenate(
        [v_all[BS:].reshape(NCH, BS, D),
         v_all[:MB].reshape(NCH, BS, D)], axis=1)
    bo = lax.dot_general(
        p.astype(jnp.bfloat16), v_cat.astype(jnp.bfloat16),
        (((2,), (1,)), ((0,), (0,))),
        preferred_element_type=jnp.float32)
    so_ref[0] = bo.reshape(MB, D)
    slog_ref[0] = jnp.broadcast_to(
        (jnp.log(s) + m).reshape(MB, 1), (MB, 16))


def _pedge_map(g):
    c0 = g * NCH
    return jnp.where(c0 % NCHUNK == 0, c0 + NCHUNK - 1, c0 - 1)


def _attention(sqk, sv, st, st_prev):
    rows = sqk.reshape(GSTEPS, MB, D)
    vrows = sv.reshape(GSTEPS, MB, D)
    id_lane_c = st.reshape(G, 1, BS)
    id_lane_p = st_prev.reshape(G, 1, BS)
    id_col = st.reshape(GSTEPS, MB, 1)
    return pl.pallas_call(
        _attn_body,
        grid=(GSTEPS,),
        in_specs=[
            pl.BlockSpec((1, MB, D), lambda g: (g, 0, 0)),
            pl.BlockSpec((1, BS, D), lambda g: (_pedge_map(g), 0, 0)),
            pl.BlockSpec((NCH, 1, BS), lambda g: (g, 0, 0)),
            pl.BlockSpec((NCH, 1, BS), lambda g: (g, 0, 0)),
            pl.BlockSpec((1, MB, 1), lambda g: (g, 0, 0)),
            pl.BlockSpec((1, MB, D), lambda g: (g, 0, 0)),
            pl.BlockSpec((1, BS, D), lambda g: (_pedge_map(g), 0, 0)),
        ],
        out_specs=[
            pl.BlockSpec((1, MB, D), lambda g: (g, 0, 0)),
            pl.BlockSpec((1, MB, 16), lambda g: (g, 0, 0)),
        ],
        out_shape=[
            jax.ShapeDtypeStruct((GSTEPS, MB, D), jnp.float32),
            jax.ShapeDtypeStruct((GSTEPS, MB, 16), jnp.float32),
        ],
    )(rows, sqk, id_lane_c, id_lane_p, id_col, vrows, sv)


_TB = 256


def _combine_body(o_ref, l_ref, out_ref):
    l = l_ref[0][:, :, 0]
    m = jnp.max(l, axis=1, keepdims=True)
    e = jnp.exp(l - m)
    w = e / jnp.sum(e, axis=1, keepdims=True)
    expand = jnp.where(
        lax.broadcasted_iota(jnp.int32, (H, H * D), 1) // D
        == lax.broadcasted_iota(jnp.int32, (H, H * D), 0),
        1.0, 0.0)
    wb = jnp.dot(w, expand, preferred_element_type=jnp.float32)
    acc = jnp.zeros((_TB, D), jnp.float32)
    for h in range(H):
        acc = acc + o_ref[0, h] * wb[:, h * D:(h + 1) * D]
    out_ref[0] = acc


def _combine(o_tok, l_tok):
    return pl.pallas_call(
        _combine_body,
        grid=(B, T // _TB),
        in_specs=[
            pl.BlockSpec((1, H, _TB, D), lambda b, t: (b, 0, t, 0)),
            pl.BlockSpec((1, _TB, H, 16), lambda b, t: (b, t, 0, 0)),
        ],
        out_specs=pl.BlockSpec((1, _TB, D), lambda b, t: (b, t, 0)),
        out_shape=jax.ShapeDtypeStruct((B, T, D), jnp.float32),
    )(o_tok, l_tok)


@jax.jit
def _run(qk, v):
    rot = jax.random.normal(
        jax.random.key(42), (1, D, H, NB // 2), dtype=jnp.float32)
    rmat_t = rot[0].reshape(D, H * (NB // 2)).T

    keys = _hash_keys(qk.transpose(0, 2, 1), rmat_t).reshape(B, S)
    sort_idx = jnp.argsort(keys, axis=-1).astype(jnp.int32)
    st = (sort_idx % T).astype(jnp.int32)

    gather_idx = (st + jnp.arange(B, dtype=jnp.int32)[:, None] * T
                  ).reshape(ROWS)
    sc_gather = _make_sc_gather()
    sqk, sv = sc_gather(gather_idx, qk.reshape(B * T, D), v.reshape(B * T, D))

    st_prev = jnp.roll(st, BS, axis=1)
    so, slog = _attention(
        sqk.reshape(G, BS, D), sv.reshape(G, BS, D),
        st.reshape(G, BS), st_prev.reshape(G, BS))

    harr = (jnp.arange(S, dtype=jnp.int32) // T)[None, :]
    boff = jnp.arange(B, dtype=jnp.int32)[:, None] * S
    dst_o = (harr * T + st + boff).reshape(NW, RPW // CH, CH)
    dst_l = (st * H + harr + boff).reshape(NW, RPW // CH, CH)
    sc_unsort = _make_sc_unsort()
    o_tok, l_tok = sc_unsort(
        dst_o, dst_l, so.reshape(ROWS, D), slog.reshape(ROWS, 16))

    return _combine(o_tok.reshape(B, H, T, D), l_tok.reshape(B, T, H, 16))


def kernel(qk, v):
    return _run(qk, v)
